# Initial kernel scaffold; baseline (speedup 1.0000x reference)
import jax
import jax.numpy as jnp
from jax.experimental import pallas as pl


def kernel(h_id_tensor, r_id_tensor, t_id_tensor, q_emb, entity_embs, num_non_text_entities, relation_embs, topic_entity_one_hot, non_text_emb, W1, b1, W2, b2):
    raise NotImplementedError("write your pallas kernel here")


# speedup vs baseline: 4.9188x; 4.9188x over previous
import functools

import jax
import jax.numpy as jnp
from jax import lax
from jax.experimental import pallas as pl
from jax.experimental.pallas import tpu as pltpu
from jax.experimental.pallas import tpu_sc as plsc

N = 10000
NPAD = 10240
E = 320000
EMB = 128
RPAD = 512
KB = 80
EROWS = 4096
EPAD = EROWS * KB
NC, NS = 2, 16
ROWS_PER_TILE = NPAD // NS
EROWS_PER_TILE = EROWS // NS
CHUNK = 32
NCHUNK = EROWS_PER_TILE // CHUNK
SCORE_ROWS_PER_W = EROWS // (NC * NS)


def _dde_direction(sid, src2d, dst2d, feat0_hbm, acc1_out, acc2_out, feat2,
                   acc1_sh, acc2_sh, sidx, didx, vbuf, dstage, fstage):
  iota = lax.iota(jnp.int32, 16)
  zv = jnp.zeros((16,), jnp.float32)
  r4 = iota >> 3
  c4 = iota & 7

  for k in range(KB // 2):
    plsc.store_scatter(vbuf, [k * 2 + r4, c4], zv)
  base = sid * ROWS_PER_TILE
  for k in range(ROWS_PER_TILE // KB):
    pltpu.sync_copy(vbuf, acc1_sh.at[pl.ds(base + k * KB, KB)])
    pltpu.sync_copy(vbuf, acc2_sh.at[pl.ds(base + k * KB, KB)])
  plsc.subcore_barrier()

  def pass_over_edges(gather_hbm, acc_sh):
    def chunk_body(c, _):
      crow = sid * EROWS_PER_TILE + c * CHUNK
      pltpu.sync_copy(src2d.at[pl.ds(crow, CHUNK)], sidx)
      pltpu.sync_copy(dst2d.at[pl.ds(crow, CHUNK)], didx)

      def blk(j, _):
        pltpu.sync_copy(gather_hbm.at[sidx.at[j]], vbuf)
        pltpu.sync_copy(vbuf, acc_sh.at[didx.at[j]], add=True)
        return 0

      lax.fori_loop(0, CHUNK, blk, 0)
      return 0

    lax.fori_loop(0, NCHUNK, chunk_body, 0)

  pass_over_edges(feat0_hbm, acc1_sh)
  plsc.subcore_barrier()

  pltpu.sync_copy(acc1_sh.at[pl.ds(base, ROWS_PER_TILE)], dstage)
  two = jnp.full((16,), 2, jnp.int32)

  def div_body(i, _):
    fl = i * 16 + iota
    r = fl >> 3
    c = fl & 7
    v = plsc.load_gather(dstage, [r, c])
    cnt = plsc.load_gather(dstage, [r, two])
    plsc.store_scatter(fstage, [r, c], v / jnp.maximum(cnt, 1.0))
    return 0

  lax.fori_loop(0, ROWS_PER_TILE * 8 // 16, div_body, 0)
  pltpu.sync_copy(dstage, acc1_out.at[pl.ds(base, ROWS_PER_TILE)])
  pltpu.sync_copy(fstage, feat2.at[pl.ds(base, ROWS_PER_TILE)])
  plsc.subcore_barrier()

  pass_over_edges(feat2, acc2_sh)
  plsc.subcore_barrier()

  pltpu.sync_copy(acc2_sh.at[pl.ds(base, ROWS_PER_TILE)], dstage)
  pltpu.sync_copy(dstage, acc2_out.at[pl.ds(base, ROWS_PER_TILE)])


def _dde_body(feat0_hbm, h2d, t2d,
              accf1, accf2, accr1, accr2, feat2f, feat2r,
              acc1_sh, acc2_sh, sidx, didx, vbuf, dstage, fstage):
  cid = lax.axis_index("c")
  sid = lax.axis_index("s")
  args = (acc1_sh, acc2_sh, sidx, didx, vbuf, dstage, fstage)
  pl.when(cid == 0)(lambda: _dde_direction(
      sid, h2d, t2d, feat0_hbm, accf1, accf2, feat2f, *args))
  pl.when(cid == 1)(lambda: _dde_direction(
      sid, t2d, h2d, feat0_hbm, accr1, accr2, feat2r, *args))


def _projection_body(he0, topic, af1, af2, ar1, ar2,
                     wa, wtop, w4f1, w4f2, w4r1, w4r2,
                     relp, wr, q8, wq, b1r,
                     p_out, c_out):
  f32 = jnp.float32
  dot = functools.partial(jnp.dot, preferred_element_type=f32)
  cf = jnp.maximum(af1[:, 2:3], 1.0)
  cr = jnp.maximum(ar1[:, 2:3], 1.0)
  acc = dot(he0[...], wa[...])
  acc += dot(topic[...], wtop[...])
  acc += dot(af1[...] / cf, w4f1[...])
  acc += dot(af2[...] / cf, w4f2[...])
  acc += dot(ar1[...] / cr, w4r1[...])
  acc += dot(ar2[...] / cr, w4r2[...])
  p_out[...] = acc

  @pl.when(pl.program_id(0) == 0)
  def _():
    c_out[...] = dot(relp[...], wr[...]) + dot(q8[...], wq[...])[0:1, :] + b1r[...]


def _score_body(th, tr, tt, h2d, r2d, t2d, w2in, b2in, out_hbm,
                hidx, ridx, tidx, bufh, bufr, buft, w2v, b2v, ostage,
                sem0, sem1, sem2):
  cid = lax.axis_index("c")
  sid = lax.axis_index("s")
  wid = cid * NS + sid
  iota = lax.iota(jnp.int32, 16)
  pltpu.sync_copy(w2in, w2v)
  pltpu.sync_copy(b2in, b2v)
  acc0 = jnp.where(iota == 0, b2v[...], 0.0)

  def chunk_body(c, _):
    crow = wid * SCORE_ROWS_PER_W + c * CHUNK
    pltpu.sync_copy(h2d.at[pl.ds(crow, CHUNK)], hidx)
    pltpu.sync_copy(r2d.at[pl.ds(crow, CHUNK)], ridx)
    pltpu.sync_copy(t2d.at[pl.ds(crow, CHUNK)], tidx)

    def block_body(j, _):
      row = crow + j
      ch = pltpu.async_copy(th.at[hidx.at[j]], bufh, sem0)
      cr = pltpu.async_copy(tr.at[ridx.at[j]], bufr, sem1)
      ct = pltpu.async_copy(tt.at[tidx.at[j]], buft, sem2)
      ch.wait()
      cr.wait()
      ct.wait()

      def group_body(g, _):
        def lane_body(l, resvec):
          e = g * 16 + l
          acc = acc0
          for d in range(EMB // 16):
            sl = pl.ds(d * 16, 16)
            v = bufh[e, sl] + bufr[e, sl] + buft[e, sl]
            acc = acc + jnp.maximum(v, 0.0) * w2v[sl]
          return jnp.where(iota == l, jnp.sum(acc), resvec)

        res = lax.fori_loop(0, 16, lane_body, jnp.zeros((16,), jnp.float32))
        ostage[pl.ds(g * 16, 16)] = res
        return 0

      lax.fori_loop(0, KB // 16, group_body, 0)
      pltpu.sync_copy(ostage, out_hbm.at[pl.ds(row * KB, KB)])
      return 0

    lax.fori_loop(0, CHUNK, block_body, 0)
    return 0

  lax.fori_loop(0, SCORE_ROWS_PER_W // CHUNK, chunk_body, 0)


def kernel(h_id_tensor, r_id_tensor, t_id_tensor, q_emb, entity_embs,
           num_non_text_entities, relation_embs, topic_entity_one_hot,
           non_text_emb, W1, b1, W2, b2):
  f32 = jnp.float32
  num_nodes = topic_entity_one_hot.shape[0]
  n_non_text = num_nodes - entity_embs.shape[0]
  non_text_residual = (jnp.asarray(num_non_text_entities) - n_non_text).astype(f32)
  he0 = jnp.concatenate([
      entity_embs,
      jnp.broadcast_to(non_text_emb, (n_non_text, EMB)) + non_text_residual,
  ], axis=0)

  feat0 = jnp.concatenate([
      topic_entity_one_hot,
      jnp.ones((num_nodes, 1), f32),
      jnp.zeros((num_nodes, 5), f32),
  ], axis=1)
  feat0 = jnp.pad(feat0, ((0, NPAD - num_nodes), (0, 0)))
  npi = jnp.full((EPAD - E,), NPAD - 1, jnp.int32)
  h2d = jnp.concatenate([h_id_tensor, npi]).reshape(EROWS, KB)
  t2d = jnp.concatenate([t_id_tensor, npi]).reshape(EROWS, KB)
  r2d = jnp.concatenate([r_id_tensor, jnp.zeros((EPAD - E,), jnp.int32)]
                        ).reshape(EROWS, KB)

  mesh = plsc.VectorSubcoreMesh(core_axis_name="c", subcore_axis_name="s",
                                num_cores=NC, num_subcores=NS)
  acc8 = jax.ShapeDtypeStruct((NPAD, 8), f32)
  dde = pl.kernel(
      _dde_body,
      out_type=(acc8,) * 6,
      mesh=mesh,
      scratch_types=[
          pltpu.VMEM_SHARED((NPAD, 8), f32),
          pltpu.VMEM_SHARED((NPAD, 8), f32),
          pltpu.VMEM((CHUNK, KB), jnp.int32),
          pltpu.VMEM((CHUNK, KB), jnp.int32),
          pltpu.VMEM((KB, 8), f32),
          pltpu.VMEM((ROWS_PER_TILE, 8), f32),
          pltpu.VMEM((ROWS_PER_TILE, 8), f32),
      ],
      compiler_params=pltpu.CompilerParams(needs_layout_passes=False,
                                           use_tc_tiling_on_sc=False),
  )
  accf1, accf2, accr1, accr2, _, _ = dde(feat0, h2d, t2d)
  w1q, w1h, w1r, w1t = W1[0:128], W1[128:266], W1[266:394], W1[394:532]
  z6 = jnp.zeros((6, 2 * EMB), f32)

  def hs(lo, hi):
    return jnp.concatenate([w1h[lo:hi], w1t[lo:hi]], axis=1)

  wa = hs(0, 128)
  wtop = hs(128, 130)
  w4f1 = jnp.concatenate([hs(130, 132), z6], axis=0)
  w4f2 = jnp.concatenate([hs(132, 134), z6], axis=0)
  w4r1 = jnp.concatenate([hs(134, 136), z6], axis=0)
  w4r2 = jnp.concatenate([hs(136, 138), z6], axis=0)
  relp = jnp.pad(relation_embs, ((0, RPAD - relation_embs.shape[0]), (0, 0)))
  q8 = jnp.pad(q_emb, ((0, 7), (0, 0)))
  b1r = b1.reshape(1, EMB)

  mblk = 1000
  grid = N // mblk
  blk = lambda r, c: pl.BlockSpec((r, c), lambda i: (i, 0))
  fix = lambda r, c: pl.BlockSpec((r, c), lambda i: (0, 0))
  p_all, c_tab = pl.pallas_call(
      _projection_body,
      grid=(grid,),
      in_specs=[
          blk(mblk, EMB), blk(mblk, 2),
          blk(mblk, 8), blk(mblk, 8), blk(mblk, 8), blk(mblk, 8),
          fix(EMB, 2 * EMB), fix(2, 2 * EMB),
          fix(8, 2 * EMB), fix(8, 2 * EMB), fix(8, 2 * EMB), fix(8, 2 * EMB),
          fix(RPAD, EMB), fix(EMB, EMB), fix(8, EMB), fix(EMB, EMB),
          fix(1, EMB),
      ],
      out_specs=[blk(mblk, 2 * EMB), fix(RPAD, EMB)],
      out_shape=[
          jax.ShapeDtypeStruct((N, 2 * EMB), f32),
          jax.ShapeDtypeStruct((RPAD, EMB), f32),
      ],
  )(he0, topic_entity_one_hot,
    accf1[:N], accf2[:N], accr1[:N], accr2[:N],
    wa, wtop, w4f1, w4f2, w4r1, w4r2,
    relp, w1r, q8, w1q, b1r)

  zrows = jnp.zeros((NPAD - N, EMB), f32)
  th = jnp.concatenate([p_all[:, 0:EMB], zrows], axis=0)
  tt = jnp.concatenate([p_all[:, EMB:2 * EMB], zrows], axis=0)

  score = pl.kernel(
      _score_body,
      out_type=jax.ShapeDtypeStruct((EPAD,), f32),
      mesh=mesh,
      scratch_types=[
          pltpu.VMEM((CHUNK, KB), jnp.int32),
          pltpu.VMEM((CHUNK, KB), jnp.int32),
          pltpu.VMEM((CHUNK, KB), jnp.int32),
          pltpu.VMEM((KB, EMB), f32),
          pltpu.VMEM((KB, EMB), f32),
          pltpu.VMEM((KB, EMB), f32),
          pltpu.VMEM((EMB,), f32),
          pltpu.VMEM((16,), f32),
          pltpu.VMEM((KB,), f32),
          pltpu.SemaphoreType.DMA,
          pltpu.SemaphoreType.DMA,
          pltpu.SemaphoreType.DMA,
      ],
      compiler_params=pltpu.CompilerParams(needs_layout_passes=False),
  )
  out = score(th, c_tab, tt, h2d, r2d, t2d,
              W2.reshape(EMB), jnp.broadcast_to(b2, (16,)))
  return out[:E].reshape(E, 1)


# speedup vs baseline: 7.0381x; 1.4309x over previous
import functools

import jax
import jax.numpy as jnp
from jax import lax
from jax.experimental import pallas as pl
from jax.experimental.pallas import tpu as pltpu
from jax.experimental.pallas import tpu_sc as plsc

N = 10000
NPAD = 10240
E = 320000
EMB = 128
RPAD = 512
EPAD = 327680
NC, NS = 2, 16
ROWS_PER_TILE = NPAD // NS
EDGES_PER_TILE = EPAD // NS
LD = 2560
NRB = EDGES_PER_TILE // LD
EDGES_PER_W = EPAD // (NC * NS)
LS = 128
NBLK = EDGES_PER_W // LS


def _dde_direction(sid, src1d, dst1d, feat0_hbm, acc1_out, acc2_out, feat2,
                   acc1_sh, acc2_sh, vb0, vb1, sx0, sx1, dx0, dx1,
                   dstage, fstage, zrows, gs0, gs1, ss0, ss1):
  iota = lax.iota(jnp.int32, 16)
  base = sid * ROWS_PER_TILE
  ebase = sid * EDGES_PER_TILE

  pltpu.sync_copy(feat0_hbm.at[pl.ds(NPAD - ROWS_PER_TILE, ROWS_PER_TILE)],
                  zrows)
  pltpu.sync_copy(zrows, acc1_sh.at[pl.ds(base, ROWS_PER_TILE)])
  pltpu.sync_copy(zrows, acc2_sh.at[pl.ds(base, ROWS_PER_TILE)])
  plsc.subcore_barrier()

  parity = [(vb0, sx0, dx0, gs0, ss0), (vb1, sx1, dx1, gs1, ss1)]

  def pass_over_edges(gather_hbm, acc_sh):
    gdesc = [None, None]
    sdesc = [None, None]
    for j in range(NRB):
      p = j & 1
      vb, sx, dx, gs, ss = parity[p]
      if sdesc[p] is not None:
        sdesc[p].wait()
      pltpu.sync_copy(src1d.at[pl.ds(ebase + j * LD, LD)], sx)
      pltpu.sync_copy(dst1d.at[pl.ds(ebase + j * LD, LD)], dx)
      gdesc[p] = pltpu.async_copy(gather_hbm.at[sx], vb, gs)
      if j >= 1:
        q = 1 - p
        gdesc[q].wait()
        sdesc[q] = pltpu.async_copy(parity[q][0], acc_sh.at[parity[q][2]],
                                    parity[q][4], add=True)
    lp = (NRB - 1) & 1
    gdesc[lp].wait()
    sdesc[lp] = pltpu.async_copy(parity[lp][0], acc_sh.at[parity[lp][2]],
                                 parity[lp][4], add=True)
    sdesc[0].wait()
    sdesc[1].wait()

  pass_over_edges(feat0_hbm, acc1_sh)
  plsc.subcore_barrier()

  pltpu.sync_copy(acc1_sh.at[pl.ds(base, ROWS_PER_TILE)], dstage)
  two = jnp.full((16,), 2, jnp.int32)

  def div_body(i, _):
    fl = i * 16 + iota
    r = fl >> 3
    c = fl & 7
    v = plsc.load_gather(dstage, [r, c])
    cnt = plsc.load_gather(dstage, [r, two])
    plsc.store_scatter(fstage, [r, c], v / jnp.maximum(cnt, 1.0))
    return 0

  lax.fori_loop(0, ROWS_PER_TILE * 8 // 16, div_body, 0)
  pltpu.sync_copy(dstage, acc1_out.at[pl.ds(base, ROWS_PER_TILE)])
  pltpu.sync_copy(fstage, feat2.at[pl.ds(base, ROWS_PER_TILE)])
  plsc.subcore_barrier()

  pass_over_edges(feat2, acc2_sh)
  plsc.subcore_barrier()

  pltpu.sync_copy(acc2_sh.at[pl.ds(base, ROWS_PER_TILE)], dstage)
  pltpu.sync_copy(dstage, acc2_out.at[pl.ds(base, ROWS_PER_TILE)])


def _dde_body(feat0_hbm, h1d, t1d,
              accf1, accf2, accr1, accr2, feat2f, feat2r,
              acc1_sh, acc2_sh, vb0, vb1, sx0, sx1, dx0, dx1,
              dstage, fstage, zrows, gs0, gs1, ss0, ss1):
  cid = lax.axis_index("c")
  sid = lax.axis_index("s")
  args = (acc1_sh, acc2_sh, vb0, vb1, sx0, sx1, dx0, dx1,
          dstage, fstage, zrows, gs0, gs1, ss0, ss1)
  pl.when(cid == 0)(lambda: _dde_direction(
      sid, h1d, t1d, feat0_hbm, accf1, accf2, feat2f, *args))
  pl.when(cid == 1)(lambda: _dde_direction(
      sid, t1d, h1d, feat0_hbm, accr1, accr2, feat2r, *args))


def _projection_body(he0, topic, af1, af2, ar1, ar2,
                     wa, wtop, w8f1, w8f2, w8r1, w8r2,
                     relp, wr, q8, wq, b1r,
                     p_out, c_out):
  f32 = jnp.float32
  dot = functools.partial(jnp.dot, preferred_element_type=f32)
  cf = jnp.maximum(af1[:, 2:3], 1.0)
  cr = jnp.maximum(ar1[:, 2:3], 1.0)
  acc = dot(he0[...], wa[...])
  acc += dot(topic[...], wtop[...])
  acc += dot(af1[...] / cf, w8f1[...])
  acc += dot(af2[...] / cf, w8f2[...])
  acc += dot(ar1[...] / cr, w8r1[...])
  acc += dot(ar2[...] / cr, w8r2[...])
  p_out[...] = acc

  @pl.when(pl.program_id(0) == 0)
  def _():
    c_out[...] = dot(relp[...], wr[...]) + dot(q8[...], wq[...])[0:1, :] + b1r[...]


def _score_body(th, tr, tt, h1d, r1d, t1d, w2in, b2in, out_hbm,
                hidx, ridx, tidx, bh0, bh1, br0, br1, bt0, bt1,
                w2v, b2v, ostage,
                sh0, sh1, sr0, sr1, st0, st1):
  cid = lax.axis_index("c")
  sid = lax.axis_index("s")
  wid = cid * NS + sid
  wbase = wid * EDGES_PER_W
  iota = lax.iota(jnp.int32, 16)
  pltpu.sync_copy(w2in, w2v)
  pltpu.sync_copy(b2in, b2v)
  pltpu.sync_copy(h1d.at[pl.ds(wbase, EDGES_PER_W)], hidx)
  pltpu.sync_copy(r1d.at[pl.ds(wbase, EDGES_PER_W)], ridx)
  pltpu.sync_copy(t1d.at[pl.ds(wbase, EDGES_PER_W)], tidx)
  acc0 = jnp.where(iota == 0, b2v[...], 0.0)
  bufs = [(bh0, br0, bt0, sh0, sr0, st0), (bh1, br1, bt1, sh1, sr1, st1)]

  def issue(j, p):
    bh, br, bt, semh, semr, semt = bufs[p]
    sl = pl.ds(j * LS, LS)
    pltpu.async_copy(th.at[hidx.at[sl]], bh, semh)
    pltpu.async_copy(tr.at[ridx.at[sl]], br, semr)
    pltpu.async_copy(tt.at[tidx.at[sl]], bt, semt)

  issue(0, 0)
  issue(1, 1)

  def pair_body(j2, _):
    for p in (0, 1):
      j = j2 * 2 + p
      bh, br, bt, semh, semr, semt = bufs[p]
      sl = pl.ds(j * LS, LS)
      pltpu.make_async_copy(th.at[hidx.at[sl]], bh, semh).wait()
      pltpu.make_async_copy(tr.at[ridx.at[sl]], br, semr).wait()
      pltpu.make_async_copy(tt.at[tidx.at[sl]], bt, semt).wait()

      def group_body(g, _):
        def lane_body(l, resvec):
          e = g * 16 + l
          acc = acc0
          for d in range(EMB // 16):
            dsl = pl.ds(d * 16, 16)
            v = bh[e, dsl] + br[e, dsl] + bt[e, dsl]
            acc = acc + jnp.maximum(v, 0.0) * w2v[dsl]
          return jnp.where(iota == l, jnp.sum(acc), resvec)

        res = lax.fori_loop(0, 16, lane_body, jnp.zeros((16,), jnp.float32))
        ostage[pl.ds(g * 16, 16)] = res
        return 0

      lax.fori_loop(0, LS // 16, group_body, 0)
      pltpu.sync_copy(ostage, out_hbm.at[pl.ds(wbase + j * LS, LS)])
      pl.when(j + 2 < NBLK)(lambda: issue(j + 2, p))
    return 0

  lax.fori_loop(0, NBLK // 2, pair_body, 0)


def kernel(h_id_tensor, r_id_tensor, t_id_tensor, q_emb, entity_embs,
           num_non_text_entities, relation_embs, topic_entity_one_hot,
           non_text_emb, W1, b1, W2, b2):
  f32 = jnp.float32
  num_nodes = topic_entity_one_hot.shape[0]
  n_non_text = num_nodes - entity_embs.shape[0]
  non_text_residual = (jnp.asarray(num_non_text_entities) - n_non_text).astype(f32)
  he0 = jnp.concatenate([
      entity_embs,
      jnp.broadcast_to(non_text_emb, (n_non_text, EMB)) + non_text_residual,
  ], axis=0)

  feat0 = jnp.concatenate([
      topic_entity_one_hot,
      jnp.ones((num_nodes, 1), f32),
      jnp.zeros((num_nodes, 5), f32),
  ], axis=1)
  feat0 = jnp.pad(feat0, ((0, NPAD - num_nodes), (0, 0)))
  npi = jnp.full((EPAD - E,), NPAD - 1, jnp.int32)
  h1d = jnp.concatenate([h_id_tensor, npi])
  t1d = jnp.concatenate([t_id_tensor, npi])
  r1d = jnp.concatenate([r_id_tensor, jnp.zeros((EPAD - E,), jnp.int32)])

  mesh = plsc.VectorSubcoreMesh(core_axis_name="c", subcore_axis_name="s",
                                num_cores=NC, num_subcores=NS)
  acc8 = jax.ShapeDtypeStruct((NPAD, 8), f32)
  dde = pl.kernel(
      _dde_body,
      out_type=(acc8,) * 6,
      mesh=mesh,
      scratch_types=[
          pltpu.VMEM_SHARED((NPAD, 8), f32),
          pltpu.VMEM_SHARED((NPAD, 8), f32),
          pltpu.VMEM((LD, 8), f32),
          pltpu.VMEM((LD, 8), f32),
          pltpu.VMEM((LD,), jnp.int32),
          pltpu.VMEM((LD,), jnp.int32),
          pltpu.VMEM((LD,), jnp.int32),
          pltpu.VMEM((LD,), jnp.int32),
          pltpu.VMEM((ROWS_PER_TILE, 8), f32),
          pltpu.VMEM((ROWS_PER_TILE, 8), f32),
          pltpu.VMEM((ROWS_PER_TILE, 8), f32),
          pltpu.SemaphoreType.DMA,
          pltpu.SemaphoreType.DMA,
          pltpu.SemaphoreType.DMA,
          pltpu.SemaphoreType.DMA,
      ],
      compiler_params=pltpu.CompilerParams(needs_layout_passes=False,
                                           use_tc_tiling_on_sc=False),
  )
  accf1, accf2, accr1, accr2, _, _ = dde(feat0, h1d, t1d)

  w1q, w1h, w1r, w1t = W1[0:128], W1[128:266], W1[266:394], W1[394:532]
  z6 = jnp.zeros((6, 2 * EMB), f32)

  def hs(lo, hi):
    return jnp.concatenate([w1h[lo:hi], w1t[lo:hi]], axis=1)

  wa = hs(0, 128)
  wtop = hs(128, 130)
  w8f1 = jnp.concatenate([hs(130, 132), z6], axis=0)
  w8f2 = jnp.concatenate([hs(132, 134), z6], axis=0)
  w8r1 = jnp.concatenate([hs(134, 136), z6], axis=0)
  w8r2 = jnp.concatenate([hs(136, 138), z6], axis=0)
  relp = jnp.pad(relation_embs, ((0, RPAD - relation_embs.shape[0]), (0, 0)))
  q8 = jnp.pad(q_emb, ((0, 7), (0, 0)))
  b1r = b1.reshape(1, EMB)

  mblk = 1000
  grid = N // mblk
  blk = lambda r, c: pl.BlockSpec((r, c), lambda i: (i, 0))
  fix = lambda r, c: pl.BlockSpec((r, c), lambda i: (0, 0))
  p_all, c_tab = pl.pallas_call(
      _projection_body,
      grid=(grid,),
      in_specs=[
          blk(mblk, EMB), blk(mblk, 2),
          blk(mblk, 8), blk(mblk, 8), blk(mblk, 8), blk(mblk, 8),
          fix(EMB, 2 * EMB), fix(2, 2 * EMB),
          fix(8, 2 * EMB), fix(8, 2 * EMB), fix(8, 2 * EMB), fix(8, 2 * EMB),
          fix(RPAD, EMB), fix(EMB, EMB), fix(8, EMB), fix(EMB, EMB),
          fix(1, EMB),
      ],
      out_specs=[blk(mblk, 2 * EMB), fix(RPAD, EMB)],
      out_shape=[
          jax.ShapeDtypeStruct((N, 2 * EMB), f32),
          jax.ShapeDtypeStruct((RPAD, EMB), f32),
      ],
  )(he0, topic_entity_one_hot,
    accf1[:N], accf2[:N], accr1[:N], accr2[:N],
    wa, wtop, w8f1, w8f2, w8r1, w8r2,
    relp, w1r, q8, w1q, b1r)

  zrows = jnp.zeros((NPAD - N, EMB), f32)
  th = jnp.concatenate([p_all[:, 0:EMB], zrows], axis=0)
  tt = jnp.concatenate([p_all[:, EMB:2 * EMB], zrows], axis=0)

  score = pl.kernel(
      _score_body,
      out_type=jax.ShapeDtypeStruct((EPAD,), f32),
      mesh=mesh,
      scratch_types=[
          pltpu.VMEM((EDGES_PER_W,), jnp.int32),
          pltpu.VMEM((EDGES_PER_W,), jnp.int32),
          pltpu.VMEM((EDGES_PER_W,), jnp.int32),
          pltpu.VMEM((LS, EMB), f32),
          pltpu.VMEM((LS, EMB), f32),
          pltpu.VMEM((LS, EMB), f32),
          pltpu.VMEM((LS, EMB), f32),
          pltpu.VMEM((LS, EMB), f32),
          pltpu.VMEM((LS, EMB), f32),
          pltpu.VMEM((EMB,), f32),
          pltpu.VMEM((16,), f32),
          pltpu.VMEM((LS,), f32),
          pltpu.SemaphoreType.DMA,
          pltpu.SemaphoreType.DMA,
          pltpu.SemaphoreType.DMA,
          pltpu.SemaphoreType.DMA,
          pltpu.SemaphoreType.DMA,
          pltpu.SemaphoreType.DMA,
      ],
      compiler_params=pltpu.CompilerParams(needs_layout_passes=False),
  )
  out = score(th, c_tab, tt, h1d, r1d, t1d,
              W2.reshape(EMB), jnp.broadcast_to(b2, (16,)))
  return out[:E].reshape(E, 1)


# speedup vs baseline: 7.4508x; 1.0586x over previous
import functools

import jax
import jax.numpy as jnp
from jax import lax
from jax.experimental import pallas as pl
from jax.experimental.pallas import tpu as pltpu
from jax.experimental.pallas import tpu_sc as plsc

N = 10000
NPAD = 10240
E = 320000
EMB = 128
RPAD = 512
EPAD = 327680
NC, NS = 2, 16
ROWS_PER_TILE = NPAD // NS
EDGES_PER_TILE = EPAD // NS
LD = 2560
NRB = EDGES_PER_TILE // LD
EDGES_PER_W = EPAD // (NC * NS)
LS = 128
NBLK = EDGES_PER_W // LS


def _dde_direction(sid, src1d, dst1d, feat0_hbm, acc1_out, acc2_out, feat2,
                   acc1_sh, acc2_sh, vb0, vb1, sx0, sx1, dx0, dx1,
                   dstage, fstage, zrows, gs0, gs1, ss0, ss1):
  iota = lax.iota(jnp.int32, 16)
  base = sid * ROWS_PER_TILE
  ebase = sid * EDGES_PER_TILE

  pltpu.sync_copy(feat0_hbm.at[pl.ds(NPAD - ROWS_PER_TILE, ROWS_PER_TILE)],
                  zrows)
  pltpu.sync_copy(zrows, acc1_sh.at[pl.ds(base, ROWS_PER_TILE)])
  pltpu.sync_copy(zrows, acc2_sh.at[pl.ds(base, ROWS_PER_TILE)])
  plsc.subcore_barrier()

  parity = [(vb0, sx0, dx0, gs0, ss0), (vb1, sx1, dx1, gs1, ss1)]

  def pass_over_edges(gather_hbm, acc_sh):
    gdesc = [None, None]
    sdesc = [None, None]
    for j in range(NRB):
      p = j & 1
      vb, sx, dx, gs, ss = parity[p]
      if sdesc[p] is not None:
        sdesc[p].wait()
      pltpu.sync_copy(src1d.at[pl.ds(ebase + j * LD, LD)], sx)
      pltpu.sync_copy(dst1d.at[pl.ds(ebase + j * LD, LD)], dx)
      gdesc[p] = pltpu.async_copy(gather_hbm.at[sx], vb, gs)
      if j >= 1:
        q = 1 - p
        gdesc[q].wait()
        sdesc[q] = pltpu.async_copy(parity[q][0], acc_sh.at[parity[q][2]],
                                    parity[q][4], add=True)
    lp = (NRB - 1) & 1
    gdesc[lp].wait()
    sdesc[lp] = pltpu.async_copy(parity[lp][0], acc_sh.at[parity[lp][2]],
                                 parity[lp][4], add=True)
    sdesc[0].wait()
    sdesc[1].wait()

  pass_over_edges(feat0_hbm, acc1_sh)
  plsc.subcore_barrier()

  pltpu.sync_copy(acc1_sh.at[pl.ds(base, ROWS_PER_TILE)], dstage)
  two = jnp.full((16,), 2, jnp.int32)

  def div_body(i, _):
    fl = i * 16 + iota
    r = fl >> 3
    c = fl & 7
    v = plsc.load_gather(dstage, [r, c])
    cnt = plsc.load_gather(dstage, [r, two])
    plsc.store_scatter(fstage, [r, c], v / jnp.maximum(cnt, 1.0))
    return 0

  lax.fori_loop(0, ROWS_PER_TILE * 8 // 16, div_body, 0)
  pltpu.sync_copy(dstage, acc1_out.at[pl.ds(base, ROWS_PER_TILE)])
  pltpu.sync_copy(fstage, feat2.at[pl.ds(base, ROWS_PER_TILE)])
  plsc.subcore_barrier()

  pass_over_edges(feat2, acc2_sh)
  plsc.subcore_barrier()

  pltpu.sync_copy(acc2_sh.at[pl.ds(base, ROWS_PER_TILE)], dstage)
  pltpu.sync_copy(dstage, acc2_out.at[pl.ds(base, ROWS_PER_TILE)])


def _dde_body(feat0_hbm, h1d, t1d,
              accf1, accf2, accr1, accr2, feat2f, feat2r,
              acc1_sh, acc2_sh, vb0, vb1, sx0, sx1, dx0, dx1,
              dstage, fstage, zrows, gs0, gs1, ss0, ss1):
  cid = lax.axis_index("c")
  sid = lax.axis_index("s")
  args = (acc1_sh, acc2_sh, vb0, vb1, sx0, sx1, dx0, dx1,
          dstage, fstage, zrows, gs0, gs1, ss0, ss1)
  pl.when(cid == 0)(lambda: _dde_direction(
      sid, h1d, t1d, feat0_hbm, accf1, accf2, feat2f, *args))
  pl.when(cid == 1)(lambda: _dde_direction(
      sid, t1d, h1d, feat0_hbm, accr1, accr2, feat2r, *args))


def _projection_body(he0, topic, af1, af2, ar1, ar2,
                     wa, wtop, w8f1, w8f2, w8r1, w8r2,
                     relp, wr, q8, wq, b1r,
                     th_out, tt_out, c_out):
  f32 = jnp.float32
  dot = functools.partial(jnp.dot, preferred_element_type=f32)
  cf = jnp.maximum(af1[:, 2:3], 1.0)
  cr = jnp.maximum(ar1[:, 2:3], 1.0)
  acc = dot(he0[...], wa[...])
  acc += dot(topic[...], wtop[...])
  acc += dot(af1[...] / cf, w8f1[...])
  acc += dot(af2[...] / cf, w8f2[...])
  acc += dot(ar1[...] / cr, w8r1[...])
  acc += dot(ar2[...] / cr, w8r2[...])
  th_out[...] = acc[:, 0:EMB]
  tt_out[...] = acc[:, EMB:2 * EMB]

  @pl.when(pl.program_id(0) == 0)
  def _():
    c_out[...] = dot(relp[...], wr[...]) + dot(q8[...], wq[...])[0:1, :] + b1r[...]


def _score_body(th, tr, tt, h1d, r1d, t1d, w2in, b2in, out_hbm,
                hidx, ridx, tidx, bh0, bh1, br0, br1, bt0, bt1,
                w2v, b2v, ost0, ost1,
                sh0, sh1, sr0, sr1, st0, st1, so0, so1):
  cid = lax.axis_index("c")
  sid = lax.axis_index("s")
  wid = cid * NS + sid
  wbase = wid * EDGES_PER_W
  iota = lax.iota(jnp.int32, 16)
  pltpu.sync_copy(w2in, w2v)
  pltpu.sync_copy(b2in, b2v)
  pltpu.sync_copy(h1d.at[pl.ds(wbase, EDGES_PER_W)], hidx)
  pltpu.sync_copy(r1d.at[pl.ds(wbase, EDGES_PER_W)], ridx)
  pltpu.sync_copy(t1d.at[pl.ds(wbase, EDGES_PER_W)], tidx)
  roff = (wid & 7) * RPAD

  def radj(i, _):
    sl = pl.ds(i * 16, 16)
    ridx[sl] = ridx[sl] + roff
    return 0

  lax.fori_loop(0, EDGES_PER_W // 16, radj, 0)
  acc0 = jnp.where(iota == 0, b2v[...], 0.0)
  bufs = [(bh0, br0, bt0, sh0, sr0, st0, ost0, so0),
          (bh1, br1, bt1, sh1, sr1, st1, ost1, so1)]

  def issue(j, p):
    bh, br, bt, semh, semr, semt, _, _ = bufs[p]
    sl = pl.ds(j * LS, LS)
    pltpu.async_copy(th.at[hidx.at[sl]], bh, semh)
    pltpu.async_copy(tr.at[ridx.at[sl]], br, semr)
    pltpu.async_copy(tt.at[tidx.at[sl]], bt, semt)

  issue(0, 0)
  issue(1, 1)

  def pair_body(j2, _):
    for p in (0, 1):
      j = j2 * 2 + p
      bh, br, bt, semh, semr, semt, ostage, semo = bufs[p]
      sl = pl.ds(j * LS, LS)
      pltpu.make_async_copy(th.at[hidx.at[sl]], bh, semh).wait()
      pltpu.make_async_copy(tr.at[ridx.at[sl]], br, semr).wait()
      pltpu.make_async_copy(tt.at[tidx.at[sl]], bt, semt).wait()
      pl.when(j >= 2)(lambda: pltpu.make_async_copy(
          ostage, out_hbm.at[pl.ds(wbase + (j - 2) * LS, LS)], semo).wait())

      def group_body(g, _):
        def lane_body(l, resvec):
          e = g * 16 + l
          acc = acc0
          for d in range(EMB // 16):
            dsl = pl.ds(d * 16, 16)
            v = bh[e, dsl] + br[e, dsl] + bt[e, dsl]
            acc = acc + jnp.maximum(v, 0.0) * w2v[dsl]
          return jnp.where(iota == l, jnp.sum(acc), resvec)

        res = lax.fori_loop(0, 16, lane_body, jnp.zeros((16,), jnp.float32))
        ostage[pl.ds(g * 16, 16)] = res
        return 0

      lax.fori_loop(0, LS // 16, group_body, 0)
      pltpu.async_copy(ostage, out_hbm.at[pl.ds(wbase + j * LS, LS)], semo)
      pl.when(j + 2 < NBLK)(lambda: issue(j + 2, p))
    return 0

  lax.fori_loop(0, NBLK // 2, pair_body, 0)
  for p in (0, 1):
    j = NBLK - 2 + p
    pltpu.make_async_copy(bufs[p][6],
                          out_hbm.at[pl.ds(wbase + j * LS, LS)],
                          bufs[p][7]).wait()


def kernel(h_id_tensor, r_id_tensor, t_id_tensor, q_emb, entity_embs,
           num_non_text_entities, relation_embs, topic_entity_one_hot,
           non_text_emb, W1, b1, W2, b2):
  f32 = jnp.float32
  num_nodes = topic_entity_one_hot.shape[0]
  n_non_text = num_nodes - entity_embs.shape[0]
  non_text_residual = (jnp.asarray(num_non_text_entities) - n_non_text).astype(f32)
  he0 = jnp.concatenate([
      entity_embs,
      jnp.broadcast_to(non_text_emb, (n_non_text, EMB)) + non_text_residual,
  ], axis=0)

  feat0 = jnp.concatenate([
      topic_entity_one_hot,
      jnp.ones((num_nodes, 1), f32),
      jnp.zeros((num_nodes, 5), f32),
  ], axis=1)
  feat0 = jnp.pad(feat0, ((0, NPAD - num_nodes), (0, 0)))
  npi = jnp.full((EPAD - E,), NPAD - 1, jnp.int32)
  h1d = jnp.concatenate([h_id_tensor, npi])
  t1d = jnp.concatenate([t_id_tensor, npi])
  r1d = jnp.concatenate([r_id_tensor, jnp.zeros((EPAD - E,), jnp.int32)])

  mesh = plsc.VectorSubcoreMesh(core_axis_name="c", subcore_axis_name="s",
                                num_cores=NC, num_subcores=NS)
  acc8 = jax.ShapeDtypeStruct((NPAD, 8), f32)
  dde = pl.kernel(
      _dde_body,
      out_type=(acc8,) * 6,
      mesh=mesh,
      scratch_types=[
          pltpu.VMEM_SHARED((NPAD, 8), f32),
          pltpu.VMEM_SHARED((NPAD, 8), f32),
          pltpu.VMEM((LD, 8), f32),
          pltpu.VMEM((LD, 8), f32),
          pltpu.VMEM((LD,), jnp.int32),
          pltpu.VMEM((LD,), jnp.int32),
          pltpu.VMEM((LD,), jnp.int32),
          pltpu.VMEM((LD,), jnp.int32),
          pltpu.VMEM((ROWS_PER_TILE, 8), f32),
          pltpu.VMEM((ROWS_PER_TILE, 8), f32),
          pltpu.VMEM((ROWS_PER_TILE, 8), f32),
          pltpu.SemaphoreType.DMA,
          pltpu.SemaphoreType.DMA,
          pltpu.SemaphoreType.DMA,
          pltpu.SemaphoreType.DMA,
      ],
      compiler_params=pltpu.CompilerParams(needs_layout_passes=False,
                                           use_tc_tiling_on_sc=False),
  )
  accf1, accf2, accr1, accr2, _, _ = dde(feat0, h1d, t1d)

  w1q, w1h, w1r, w1t = W1[0:128], W1[128:266], W1[266:394], W1[394:532]
  z6 = jnp.zeros((6, 2 * EMB), f32)

  def hs(lo, hi):
    return jnp.concatenate([w1h[lo:hi], w1t[lo:hi]], axis=1)

  wa = hs(0, 128)
  wtop = hs(128, 130)
  w8f1 = jnp.concatenate([hs(130, 132), z6], axis=0)
  w8f2 = jnp.concatenate([hs(132, 134), z6], axis=0)
  w8r1 = jnp.concatenate([hs(134, 136), z6], axis=0)
  w8r2 = jnp.concatenate([hs(136, 138), z6], axis=0)
  relp = jnp.pad(relation_embs, ((0, RPAD - relation_embs.shape[0]), (0, 0)))
  q8 = jnp.pad(q_emb, ((0, 7), (0, 0)))
  b1r = b1.reshape(1, EMB)

  mblk = 1000
  grid = N // mblk
  blk = lambda r, c: pl.BlockSpec((r, c), lambda i: (i, 0))
  fix = lambda r, c: pl.BlockSpec((r, c), lambda i: (0, 0))
  th, tt, c_tab = pl.pallas_call(
      _projection_body,
      grid=(grid,),
      in_specs=[
          blk(mblk, EMB), blk(mblk, 2),
          blk(mblk, 8), blk(mblk, 8), blk(mblk, 8), blk(mblk, 8),
          fix(EMB, 2 * EMB), fix(2, 2 * EMB),
          fix(8, 2 * EMB), fix(8, 2 * EMB), fix(8, 2 * EMB), fix(8, 2 * EMB),
          fix(RPAD, EMB), fix(EMB, EMB), fix(8, EMB), fix(EMB, EMB),
          fix(1, EMB),
      ],
      out_specs=[blk(mblk, EMB), blk(mblk, EMB), fix(RPAD, EMB)],
      out_shape=[
          jax.ShapeDtypeStruct((N, EMB), f32),
          jax.ShapeDtypeStruct((N, EMB), f32),
          jax.ShapeDtypeStruct((RPAD, EMB), f32),
      ],
  )(he0, topic_entity_one_hot,
    accf1[:N], accf2[:N], accr1[:N], accr2[:N],
    wa, wtop, w8f1, w8f2, w8r1, w8r2,
    relp, w1r, q8, w1q, b1r)


  score = pl.kernel(
      _score_body,
      out_type=jax.ShapeDtypeStruct((EPAD,), f32),
      mesh=mesh,
      scratch_types=[
          pltpu.VMEM((EDGES_PER_W,), jnp.int32),
          pltpu.VMEM((EDGES_PER_W,), jnp.int32),
          pltpu.VMEM((EDGES_PER_W,), jnp.int32),
          pltpu.VMEM((LS, EMB), f32),
          pltpu.VMEM((LS, EMB), f32),
          pltpu.VMEM((LS, EMB), f32),
          pltpu.VMEM((LS, EMB), f32),
          pltpu.VMEM((LS, EMB), f32),
          pltpu.VMEM((LS, EMB), f32),
          pltpu.VMEM((EMB,), f32),
          pltpu.VMEM((16,), f32),
          pltpu.VMEM((LS,), f32),
          pltpu.VMEM((LS,), f32),
          pltpu.SemaphoreType.DMA,
          pltpu.SemaphoreType.DMA,
          pltpu.SemaphoreType.DMA,
          pltpu.SemaphoreType.DMA,
          pltpu.SemaphoreType.DMA,
          pltpu.SemaphoreType.DMA,
          pltpu.SemaphoreType.DMA,
          pltpu.SemaphoreType.DMA,
      ],
      compiler_params=pltpu.CompilerParams(needs_layout_passes=False),
  )
  zpad = jnp.zeros((EPAD - E,), jnp.int32)
  hs1d = jnp.concatenate([h_id_tensor, zpad])
  ts1d = jnp.concatenate([t_id_tensor, zpad])
  c_rep = jnp.tile(c_tab, (8, 1))
  out = score(th, c_rep, tt, hs1d, r1d, ts1d,
              W2.reshape(EMB), jnp.broadcast_to(b2, (16,)))
  return out[:E].reshape(E, 1)
